# baseline (speedup 1.0000x reference)
import functools

import jax
import jax.numpy as jnp
from jax import lax
from jax.experimental import pallas as pl
from jax.experimental.pallas import tpu as pltpu
from jax.experimental.pallas import tpu_sc as plsc

D = 64
NC = 2
NS = 16
NW = NC * NS
C = 512
SLAB = C * D


def _make(BATCH, SEQ):
    assert SEQ == C
    q_per_w = BATCH // NW
    assert q_per_w % 2 == 0
    mesh = plsc.VectorSubcoreMesh(core_axis_name="c", subcore_axis_name="s")

    @functools.partial(
        pl.kernel,
        mesh=mesh,
        out_type=jax.ShapeDtypeStruct((BATCH, SLAB), jnp.float32),
        compiler_params=pltpu.CompilerParams(
            use_tc_tiling_on_sc=False, needs_layout_passes=False),
        scratch_types=[
            pltpu.VMEM((C,), jnp.int32),
            pltpu.VMEM((C,), jnp.int32),
            pltpu.VMEM((C, D), jnp.float32),
            pltpu.VMEM((C, D), jnp.float32),
            pltpu.VMEM((SLAB,), jnp.float32),
            pltpu.SemaphoreType.DMA,
            pltpu.SemaphoreType.DMA,
            pltpu.SemaphoreType.DMA,
        ],
    )
    def k(xf_hbm, table_hbm, out_hbm, iv0, iv1, rows0, rows1, slab,
          sg0, sg1, sw):
        wid = lax.axis_index("s") * NC + lax.axis_index("c")
        base = wid * q_per_w
        iota = lax.iota(jnp.int32, 16)

        def stage(q, iv):
            pltpu.sync_copy(xf_hbm.at[pl.ds((base + q) * C, C)], iv)

        def start_gather(iv, rows, sg):
            pltpu.async_copy(table_hbm.at[iv], rows, sg)

        def wait_gather(iv, rows, sg):
            pltpu.make_async_copy(table_hbm.at[iv], rows, sg).wait()

        def start_write(q, sw):
            pltpu.async_copy(slab, out_hbm.at[base + q], sw)

        def wait_write(sw):
            pltpu.make_async_copy(slab, out_hbm.at[base], sw).wait()

        def transpose(rows):
            @plsc.parallel_loop(0, 4 * D, unroll=8)
            def dbody(t):
                ts = t // D
                d = t % D
                doff = ts * 1024 + (d // 8) * 4096 + (d % 8) * 128
                tsr = ts * 128
                cvec = jnp.full((16,), 0, jnp.int32) + d
                for cb in range(8):
                    rvec = iota + (tsr + cb * 16)
                    v = plsc.load_gather(rows, [rvec, cvec])
                    slab[pl.ds(doff + cb * 16, 16)] = v

        stage(0, iv0)
        start_gather(iv0, rows0, sg0)

        wait_gather(iv0, rows0, sg0)
        stage(1, iv1)
        start_gather(iv1, rows1, sg1)
        transpose(rows0)
        start_write(0, sw)

        wait_gather(iv1, rows1, sg1)
        stage(2, iv0)
        start_gather(iv0, rows0, sg0)
        wait_write(sw)
        transpose(rows1)
        start_write(1, sw)

        def body(j, carry):
            i = 2 * j
            wait_gather(iv0, rows0, sg0)
            stage(i + 1, iv1)
            start_gather(iv1, rows1, sg1)
            wait_write(sw)
            transpose(rows0)
            start_write(i, sw)

            wait_gather(iv1, rows1, sg1)
            stage(jnp.minimum(i + 2, q_per_w - 1), iv0)
            start_gather(iv0, rows0, sg0)
            wait_write(sw)
            transpose(rows1)
            start_write(i + 1, sw)
            return carry

        lax.fori_loop(1, q_per_w // 2, body, 0)

        wait_gather(iv0, rows0, sg0)
        wait_write(sw)

    return k


def kernel(x, table):
    b, s = x.shape
    xf = x.reshape(b * s).astype(jnp.int32)
    out2 = _make(b, s)(xf, table)
    return (
        out2.reshape(b, 8, 4, 8, 128)
        .transpose(0, 2, 4, 1, 3)
        .reshape(b, s, D)
    )


# speedup vs baseline: 1.5700x; 1.5700x over previous
import functools

import jax
import jax.numpy as jnp
from jax import lax
from jax.experimental import pallas as pl
from jax.experimental.pallas import tpu as pltpu
from jax.experimental.pallas import tpu_sc as plsc

D = 64
NC = 2
NS = 16
NW = NC * NS
C = 512
SLAB = C * D


def _make(BATCH, SEQ):
    assert SEQ == C
    q_per_w = BATCH // NW
    assert q_per_w % 2 == 0
    mesh = plsc.VectorSubcoreMesh(core_axis_name="c", subcore_axis_name="s")

    @functools.partial(
        pl.kernel,
        mesh=mesh,
        out_type=jax.ShapeDtypeStruct((BATCH, SLAB), jnp.float32),
        compiler_params=pltpu.CompilerParams(
            use_tc_tiling_on_sc=False, needs_layout_passes=False),
        scratch_types=[
            pltpu.VMEM((C,), jnp.int32),
            pltpu.VMEM((C,), jnp.int32),
            pltpu.VMEM((C, D + 1), jnp.float32),
            pltpu.VMEM((C, D + 1), jnp.float32),
            pltpu.VMEM((SLAB,), jnp.float32),
            pltpu.SemaphoreType.DMA,
            pltpu.SemaphoreType.DMA,
            pltpu.SemaphoreType.DMA,
        ],
    )
    def k(xf_hbm, table_hbm, out_hbm, iv0, iv1, rows0, rows1, slab,
          sg0, sg1, sw):
        wid = lax.axis_index("s") * NC + lax.axis_index("c")
        base = wid * q_per_w
        iota = lax.iota(jnp.int32, 16)

        def stage(q, iv):
            pltpu.sync_copy(xf_hbm.at[pl.ds((base + q) * C, C)], iv)

        def start_gather(iv, rows, sg):
            pltpu.async_copy(table_hbm.at[iv], rows, sg)

        def wait_gather(iv, rows, sg):
            pltpu.make_async_copy(table_hbm.at[iv], rows, sg).wait()

        def start_write(q, sw):
            pltpu.async_copy(slab, out_hbm.at[base + q], sw)

        def wait_write(sw):
            pltpu.make_async_copy(slab, out_hbm.at[base], sw).wait()

        def transpose(rows):
            @plsc.parallel_loop(0, 4 * D, unroll=8)
            def dbody(t):
                ts = t // D
                d = t % D
                doff = ts * 1024 + (d // 8) * 4096 + (d % 8) * 128
                tsr = ts * 128
                cvec = jnp.full((16,), 0, jnp.int32) + d
                for cb in range(8):
                    rvec = iota + (tsr + cb * 16)
                    v = plsc.load_gather(rows, [rvec, cvec])
                    slab[pl.ds(doff + cb * 16, 16)] = v

        stage(0, iv0)
        start_gather(iv0, rows0, sg0)

        wait_gather(iv0, rows0, sg0)
        stage(1, iv1)
        start_gather(iv1, rows1, sg1)
        transpose(rows0)
        start_write(0, sw)

        wait_gather(iv1, rows1, sg1)
        stage(2, iv0)
        start_gather(iv0, rows0, sg0)
        wait_write(sw)
        transpose(rows1)
        start_write(1, sw)

        def body(j, carry):
            i = 2 * j
            wait_gather(iv0, rows0, sg0)
            stage(i + 1, iv1)
            start_gather(iv1, rows1, sg1)
            wait_write(sw)
            transpose(rows0)
            start_write(i, sw)

            wait_gather(iv1, rows1, sg1)
            stage(jnp.minimum(i + 2, q_per_w - 1), iv0)
            start_gather(iv0, rows0, sg0)
            wait_write(sw)
            transpose(rows1)
            start_write(i + 1, sw)
            return carry

        lax.fori_loop(1, q_per_w // 2, body, 0)

        wait_gather(iv0, rows0, sg0)
        wait_write(sw)

    return k


def kernel(x, table):
    b, s = x.shape
    xf = x.reshape(b * s).astype(jnp.int32)
    t65 = jnp.pad(table, ((0, 0), (0, 1)))
    out2 = _make(b, s)(xf, t65)
    return (
        out2.reshape(b, 8, 4, 8, 128)
        .transpose(0, 2, 4, 1, 3)
        .reshape(b, s, D)
    )


# speedup vs baseline: 2.1598x; 1.3757x over previous
import functools

import jax
import jax.numpy as jnp
from jax import lax
from jax.experimental import pallas as pl
from jax.experimental.pallas import tpu as pltpu
from jax.experimental.pallas import tpu_sc as plsc

D = 64
NC = 2
NS = 16
NW = NC * NS
C = 512
SLAB = C * D
RS = D + 1


def _make(BATCH, SEQ):
    assert SEQ == C
    q_per_w = BATCH // NW
    mesh = plsc.VectorSubcoreMesh(core_axis_name="c", subcore_axis_name="s")

    @functools.partial(
        pl.kernel,
        mesh=mesh,
        out_type=jax.ShapeDtypeStruct((BATCH, SLAB), jnp.float32),
        compiler_params=pltpu.CompilerParams(
            use_tc_tiling_on_sc=False, needs_layout_passes=False),
        scratch_types=[
            pltpu.VMEM((C,), jnp.int32),
            pltpu.VMEM((C, D), jnp.float32),
            pltpu.VMEM((C * RS,), jnp.float32),
            pltpu.VMEM((SLAB,), jnp.float32),
            pltpu.SemaphoreType.DMA,
            pltpu.SemaphoreType.DMA,
        ],
    )
    def k(xf_hbm, table_hbm, out_hbm, iv, rows, r65, slab, sg, sw):
        wid = lax.axis_index("s") * NC + lax.axis_index("c")
        base = wid * q_per_w
        iota = lax.iota(jnp.int32, 16)
        iota65 = iota * RS

        def stage(q):
            pltpu.sync_copy(xf_hbm.at[pl.ds((base + q) * C, C)], iv)

        def start_gather():
            pltpu.async_copy(table_hbm.at[iv], rows, sg)

        def wait_gather():
            pltpu.make_async_copy(table_hbm.at[iv], rows, sg).wait()

        def start_write(q):
            pltpu.async_copy(slab, out_hbm.at[base + q], sw)

        def wait_write():
            pltpu.make_async_copy(slab, out_hbm.at[base], sw).wait()

        def restride():
            @plsc.parallel_loop(0, C, unroll=8)
            def rbody(tok):
                o = tok * RS
                for kk in range(4):
                    r65[pl.ds(o + 16 * kk, 16)] = rows[tok, pl.ds(16 * kk, 16)]

        def transpose():
            @plsc.parallel_loop(0, 4 * D, unroll=8)
            def dbody(t):
                ts = t // D
                d = t % D
                doff = ts * 1024 + (d // 8) * 4096 + (d % 8) * 128
                rbase = ts * 128 * RS + d
                for cb in range(8):
                    idx = iota65 + (rbase + cb * 16 * RS)
                    v = plsc.load_gather(r65, [idx])
                    slab[pl.ds(doff + cb * 16, 16)] = v

        stage(0)
        start_gather()
        wait_gather()
        restride()
        stage(1)
        start_gather()
        transpose()
        start_write(0)

        def body(i, carry):
            wait_gather()
            restride()
            stage(jnp.minimum(i + 1, q_per_w - 1))
            start_gather()
            wait_write()
            transpose()
            start_write(i)
            return carry

        lax.fori_loop(1, q_per_w, body, 0)

        wait_gather()
        wait_write()

    return k


def kernel(x, table):
    b, s = x.shape
    xf = x.reshape(b * s).astype(jnp.int32)
    out2 = _make(b, s)(xf, table)
    return (
        out2.reshape(b, 8, 4, 8, 128)
        .transpose(0, 2, 4, 1, 3)
        .reshape(b, s, D)
    )
